# baseline (speedup 1.0000x reference)
import functools

import jax
import jax.numpy as jnp
from jax import lax
from jax.experimental import pallas as pl
from jax.experimental.pallas import tpu as pltpu
from jax.experimental.pallas import tpu_sc as plsc

_N = 10000
_E = 320000
_D = 128
_NC = 2
_NS = 16
_NW = _NC * _NS
_C = 128
_KA = 91
_KB = 67
_KM = max(_KA, _KB)
_NCH = _NS * (_KA + _KB)
_EPAD = _NCH * _C
_RPT = 632
_NPAD = _NS * _RPT
_RPE = _N // _NS
_BLK = 1000
_G = _N // _BLK

_f32 = jnp.float32
_i32 = jnp.int32


def _sc_mesh():
    return plsc.VectorSubcoreMesh(
        core_axis_name="c", subcore_axis_name="s",
        num_cores=_NC, num_subcores=_NS)


def _deg_kernel(dst3, ew3):
    def body(dst3_hbm, ew3_hbm, degp_hbm, dst_v, ew_v, zbuf, deg_sh):
        cid = lax.axis_index("c")
        sid = lax.axis_index("s")
        base = jnp.where(cid == 0, sid * _KA, _NS * _KA + sid * _KB)
        kc = jnp.where(cid == 0, _KA, _KB)
        r0 = pl.multiple_of(sid * _RPT, 8)

        @pl.loop(0, 40)
        def zz(t):
            zbuf[pl.ds(t * 16, 16)] = jnp.zeros((16,), _f32)

        pltpu.sync_copy(zbuf.at[pl.ds(0, _RPT)], deg_sh.at[pl.ds(r0, _RPT)])
        ld = jnp.minimum(base, _NCH - _KM)
        off = base - ld
        pltpu.sync_copy(dst3_hbm.at[pl.ds(ld, _KM), :], dst_v)
        pltpu.sync_copy(ew3_hbm.at[pl.ds(ld, _KM), :], ew_v)
        plsc.subcore_barrier()

        @pl.loop(0, kc)
        def chunk(j):
            pltpu.sync_copy(ew_v.at[off + j], deg_sh.at[dst_v.at[off + j]],
                            add=True)

        plsc.subcore_barrier()
        w0 = pl.multiple_of(cid * _NPAD + sid * _RPT, 8)
        pltpu.sync_copy(deg_sh.at[pl.ds(r0, _RPT)], zbuf.at[pl.ds(0, _RPT)])
        pltpu.sync_copy(zbuf.at[pl.ds(0, _RPT)], degp_hbm.at[pl.ds(w0, _RPT)])

    f = pl.kernel(
        body,
        out_type=jax.ShapeDtypeStruct((_NC * _NPAD,), _f32),
        mesh=_sc_mesh(),
        compiler_params=pltpu.CompilerParams(use_tc_tiling_on_sc=False),
        scratch_types=[
            pltpu.VMEM((_KM, _C), _i32),
            pltpu.VMEM((_KM, _C), _f32),
            pltpu.VMEM((640,), _f32),
            pltpu.VMEM_SHARED((_NPAD,), _f32),
        ],
    )
    return f(dst3, ew3)


def _edge_kernel(h, src3, dst3, ew3):
    def body(h_hbm, src3_hbm, dst3_hbm, ew3_hbm, sp_hbm,
             srcr, dstr, ewr, buf0, buf1, buf2, acc_sh,
             gsem0, gsem1, gsem2, ssem0, ssem1, ssem2,
             isem0, isem1, isem2, dsem0, dsem1, dsem2,
             esem0, esem1, esem2):
        cid = lax.axis_index("c")
        sid = lax.axis_index("s")
        base = jnp.where(cid == 0, sid * _KA, _NS * _KA + sid * _KB)
        kc = jnp.where(cid == 0, _KA, _KB)
        r0 = sid * _RPE
        bufs = (buf0, buf1, buf2)
        gsems = (gsem0, gsem1, gsem2)
        ssems = (ssem0, ssem1, ssem2)
        isems = (isem0, isem1, isem2)
        dsems = (dsem0, dsem1, dsem2)
        esems = (esem0, esem1, esem2)

        @pl.loop(0, _C)
        def zrow(e):
            for q in range(_D // 16):
                buf0[e, pl.ds(q * 16, 16)] = jnp.zeros((16,), _f32)

        nfull, rem = divmod(_RPE, _C)
        ncopies = nfull + (1 if rem else 0)
        for t in range(ncopies):
            rows = _C if t < nfull else rem
            pltpu.sync_copy(buf0.at[pl.ds(0, rows), :],
                            acc_sh.at[pl.ds(r0 + t * _C, rows), :])
        plsc.subcore_barrier()

        def scale(j, buf, s):
            @pl.loop(0, _C // 16)
            def grp(g):
                w16 = ewr[s, pl.ds(g * 16, 16)]
                for l in range(16):
                    w = w16[jnp.full((16,), l, _i32)]
                    e = g * 16 + l
                    for q in range(_D // 16):
                        sl = pl.ds(q * 16, 16)
                        buf[e, sl] = buf[e, sl] * w

        def start_src(j, s):
            pltpu.async_copy(src3_hbm.at[base + j], srcr.at[s], isems[s])

        def wait_src(j, s):
            pltpu.make_async_copy(src3_hbm.at[base + j], srcr.at[s],
                                  isems[s]).wait()

        def start_dst(j, s):
            pltpu.async_copy(dst3_hbm.at[base + j], dstr.at[s], dsems[s])

        def wait_dst(j, s):
            pltpu.make_async_copy(dst3_hbm.at[base + j], dstr.at[s],
                                  dsems[s]).wait()

        def start_ew(j, s):
            pltpu.async_copy(ew3_hbm.at[base + j], ewr.at[s], esems[s])

        def wait_ew(j, s):
            pltpu.make_async_copy(ew3_hbm.at[base + j], ewr.at[s],
                                  esems[s]).wait()

        def start_gather(j, s, b):
            pltpu.async_copy(h_hbm.at[srcr.at[s]], bufs[b], gsems[b])

        def wait_gather(j, s, b):
            pltpu.make_async_copy(h_hbm.at[srcr.at[s]], bufs[b],
                                  gsems[b]).wait()

        def start_scatter(j, s, b):
            pltpu.async_copy(bufs[b], acc_sh.at[dstr.at[s]], ssems[b],
                             add=True)

        def wait_scatter(j, s, b):
            pltpu.make_async_copy(bufs[b], acc_sh.at[dstr.at[s]],
                                  ssems[b]).wait()

        for j in (0, 1, 2):
            start_src(j, j)
        for j in (0, 1):
            start_dst(j, j)
            start_ew(j, j)
        wait_src(0, 0)
        start_gather(0, 0, 0)
        wait_src(1, 1)
        start_gather(1, 1, 1)

        def turn(jj, b, first=False, g2=True, s3=True, e2=True):
            b2 = (b + 2) % 3
            wait_gather(jj, b, b)
            wait_ew(jj, b)
            scale(jj, bufs[b], b)
            if e2:
                start_ew(jj + 2, b2)
            wait_dst(jj, b)
            start_scatter(jj, b, b)
            if not first:
                wait_scatter(jj - 1, b2, b2)
            if g2:
                wait_src(jj + 2, b2)
                start_gather(jj + 2, b2, b2)
                start_dst(jj + 2, b2)
            if s3:
                start_src(jj + 3, b)

        turn(0, 0, first=True)
        turn(1, 1)
        turn(2, 2)

        @pl.loop(3, kc - 4, step=3)
        def steady(j):
            for off in range(3):
                turn(j + off, off)

        turn(kc - 4, 0)
        turn(kc - 3, 1, s3=False)
        turn(kc - 2, 2, g2=False, s3=False, e2=False)
        turn(kc - 1, 0, g2=False, s3=False, e2=False)
        wait_scatter(kc - 1, 0, 0)

        plsc.subcore_barrier()
        for t in range(ncopies):
            rows = _C if t < nfull else rem
            pltpu.sync_copy(acc_sh.at[pl.ds(r0 + t * _C, rows), :],
                            buf0.at[pl.ds(0, rows), :])
            pltpu.sync_copy(buf0.at[pl.ds(0, rows), :],
                            sp_hbm.at[cid, pl.ds(r0 + t * _C, rows), :])

    f = pl.kernel(
        body,
        out_type=jax.ShapeDtypeStruct((_NC, _N, _D), _f32),
        mesh=_sc_mesh(),
        compiler_params=pltpu.CompilerParams(use_tc_tiling_on_sc=False),
        scratch_types=[
            pltpu.VMEM((3, _C), _i32),
            pltpu.VMEM((3, _C), _i32),
            pltpu.VMEM((3, _C), _f32),
            pltpu.VMEM((_C, _D), _f32),
            pltpu.VMEM((_C, _D), _f32),
            pltpu.VMEM((_C, _D), _f32),
            pltpu.VMEM_SHARED((_N, _D), _f32),
        ] + [pltpu.SemaphoreType.DMA] * 15,
    )
    return f(h, src3, dst3, ew3)


def _m1_body(x_ref, w_ref, d0_ref, d1_ref, hp_ref, dinv_ref):
    dinv = lax.rsqrt(1.0 + d0_ref[...] + d1_ref[...])
    h = jnp.dot(x_ref[...], w_ref[...], preferred_element_type=_f32)
    hp_ref[...] = h * dinv
    dinv_ref[...] = dinv


def _tc_m1(xp, W1, d0, d1):
    return pl.pallas_call(
        _m1_body,
        grid=(_G,),
        in_specs=[
            pl.BlockSpec((_BLK, _D), lambda i: (i, 0)),
            pl.BlockSpec((_D, _D), lambda i: (0, 0)),
            pl.BlockSpec((_BLK, 1), lambda i: (i, 0)),
            pl.BlockSpec((_BLK, 1), lambda i: (i, 0)),
        ],
        out_specs=[
            pl.BlockSpec((_BLK, _D), lambda i: (i, 0)),
            pl.BlockSpec((_BLK, 1), lambda i: (i, 0)),
        ],
        out_shape=[
            jax.ShapeDtypeStruct((_N, _D), _f32),
            jax.ShapeDtypeStruct((_N, 1), _f32),
        ],
    )(xp, W1, d0, d1)


def _m2_body(s_ref, hp_ref, dinv_ref, b_ref, w_ref, h2_ref):
    a = ((s_ref[0] + s_ref[1] + hp_ref[...]) * dinv_ref[...] + b_ref[...])
    a = jnp.maximum(a, 0.0)
    h2 = jnp.dot(a, w_ref[...], preferred_element_type=_f32)
    h2_ref[...] = h2 * dinv_ref[...]


def _tc_m2(s, hp, dinv, b, W2):
    return pl.pallas_call(
        _m2_body,
        grid=(_G,),
        in_specs=[
            pl.BlockSpec((_NC, _BLK, _D), lambda i: (0, i, 0)),
            pl.BlockSpec((_BLK, _D), lambda i: (i, 0)),
            pl.BlockSpec((_BLK, 1), lambda i: (i, 0)),
            pl.BlockSpec((1, _D), lambda i: (0, 0)),
            pl.BlockSpec((_D, _D), lambda i: (0, 0)),
        ],
        out_specs=pl.BlockSpec((_BLK, _D), lambda i: (i, 0)),
        out_shape=jax.ShapeDtypeStruct((_N, _D), _f32),
    )(s, hp, dinv, b, W2)


def _m3_body(s_ref, hp_ref, dinv_ref, b_ref, out_ref):
    out_ref[...] = ((s_ref[0] + s_ref[1] + hp_ref[...])
                    * dinv_ref[...] + b_ref[...])


def _tc_m3(s, hp, dinv, b):
    return pl.pallas_call(
        _m3_body,
        grid=(_G,),
        in_specs=[
            pl.BlockSpec((_NC, _BLK, _D), lambda i: (0, i, 0)),
            pl.BlockSpec((_BLK, _D), lambda i: (i, 0)),
            pl.BlockSpec((_BLK, 1), lambda i: (i, 0)),
            pl.BlockSpec((1, _D), lambda i: (0, 0)),
        ],
        out_specs=pl.BlockSpec((_BLK, _D), lambda i: (i, 0)),
        out_shape=jax.ShapeDtypeStruct((_N, _D), _f32),
    )(s, hp, dinv, b)


def kernel(x, edge_index, edge_weight, W1, b1, W2, b2):
    src = edge_index[0].astype(_i32)
    dst = edge_index[1].astype(_i32)
    ew = edge_weight.astype(_f32)
    pad = _EPAD - _E
    src3 = jnp.concatenate([src, jnp.zeros((pad,), _i32)]).reshape(_NCH, _C)
    dst3 = jnp.concatenate([dst, jnp.zeros((pad,), _i32)]).reshape(_NCH, _C)
    ew3 = jnp.concatenate([ew, jnp.zeros((pad,), _f32)]).reshape(_NCH, _C)

    degp = _deg_kernel(dst3, ew3).reshape(_NC, _NPAD)
    d0 = degp[0, :_N].reshape(_N, 1)
    d1 = degp[1, :_N].reshape(_N, 1)
    h1p, dinv = _tc_m1(x, W1, d0, d1)
    s1 = _edge_kernel(h1p, src3, dst3, ew3)
    h2p = _tc_m2(s1, h1p, dinv, b1.reshape(1, _D), W2)
    s2 = _edge_kernel(h2p, src3, dst3, ew3)
    outp = _tc_m3(s2, h2p, dinv, b2.reshape(1, _D))
    return outp


# speedup vs baseline: 1.0665x; 1.0665x over previous
import functools

import jax
import jax.numpy as jnp
from jax import lax
from jax.experimental import pallas as pl
from jax.experimental.pallas import tpu as pltpu
from jax.experimental.pallas import tpu_sc as plsc

_N = 10000
_E = 320000
_D = 128
_NC = 2
_NS = 16
_NW = _NC * _NS
_C = 128
_KA = 124
_KB = 34
_KM = max(_KA, _KB)
_NCH = _NS * (_KA + _KB)
_EPAD = _NCH * _C
_RPT = 632
_NPAD = _NS * _RPT
_RPE = _N // _NS
_BLK = 1000
_G = _N // _BLK

_f32 = jnp.float32
_i32 = jnp.int32


def _sc_mesh():
    return plsc.VectorSubcoreMesh(
        core_axis_name="c", subcore_axis_name="s",
        num_cores=_NC, num_subcores=_NS)


def _deg_kernel(dst3, ew3):
    def body(dst3_hbm, ew3_hbm, degp_hbm, dst_v, ew_v, zbuf, deg_sh):
        cid = lax.axis_index("c")
        sid = lax.axis_index("s")
        base = jnp.where(cid == 0, sid * _KA, _NS * _KA + sid * _KB)
        kc = jnp.where(cid == 0, _KA, _KB)
        r0 = pl.multiple_of(sid * _RPT, 8)

        @pl.loop(0, 40)
        def zz(t):
            zbuf[pl.ds(t * 16, 16)] = jnp.zeros((16,), _f32)

        pltpu.sync_copy(zbuf.at[pl.ds(0, _RPT)], deg_sh.at[pl.ds(r0, _RPT)])
        ld = jnp.minimum(base, _NCH - _KM)
        off = base - ld
        pltpu.sync_copy(dst3_hbm.at[pl.ds(ld, _KM), :], dst_v)
        pltpu.sync_copy(ew3_hbm.at[pl.ds(ld, _KM), :], ew_v)
        plsc.subcore_barrier()

        @pl.loop(0, kc)
        def chunk(j):
            pltpu.sync_copy(ew_v.at[off + j], deg_sh.at[dst_v.at[off + j]],
                            add=True)

        plsc.subcore_barrier()
        w0 = pl.multiple_of(cid * _NPAD + sid * _RPT, 8)
        pltpu.sync_copy(deg_sh.at[pl.ds(r0, _RPT)], zbuf.at[pl.ds(0, _RPT)])
        pltpu.sync_copy(zbuf.at[pl.ds(0, _RPT)], degp_hbm.at[pl.ds(w0, _RPT)])

    f = pl.kernel(
        body,
        out_type=jax.ShapeDtypeStruct((_NC * _NPAD,), _f32),
        mesh=_sc_mesh(),
        compiler_params=pltpu.CompilerParams(use_tc_tiling_on_sc=False),
        scratch_types=[
            pltpu.VMEM((_KM, _C), _i32),
            pltpu.VMEM((_KM, _C), _f32),
            pltpu.VMEM((640,), _f32),
            pltpu.VMEM_SHARED((_NPAD,), _f32),
        ],
    )
    return f(dst3, ew3)


def _edge_kernel(h, src3, dst3, ew3):
    def body(h_hbm, src3_hbm, dst3_hbm, ew3_hbm, sp_hbm,
             srcr, dstr, ewr, buf0, buf1, buf2, acc_sh,
             gsem0, gsem1, gsem2, ssem0, ssem1, ssem2,
             isem0, isem1, isem2, dsem0, dsem1, dsem2,
             esem0, esem1, esem2):
        cid = lax.axis_index("c")
        sid = lax.axis_index("s")
        base = jnp.where(cid == 0, sid * _KA, _NS * _KA + sid * _KB)
        kc = jnp.where(cid == 0, _KA, _KB)
        r0 = sid * _RPE
        bufs = (buf0, buf1, buf2)
        gsems = (gsem0, gsem1, gsem2)
        ssems = (ssem0, ssem1, ssem2)
        isems = (isem0, isem1, isem2)
        dsems = (dsem0, dsem1, dsem2)
        esems = (esem0, esem1, esem2)

        @pl.loop(0, _C)
        def zrow(e):
            for q in range(_D // 16):
                buf0[e, pl.ds(q * 16, 16)] = jnp.zeros((16,), _f32)

        nfull, rem = divmod(_RPE, _C)
        ncopies = nfull + (1 if rem else 0)
        for t in range(ncopies):
            rows = _C if t < nfull else rem
            pltpu.sync_copy(buf0.at[pl.ds(0, rows), :],
                            acc_sh.at[pl.ds(r0 + t * _C, rows), :])
        plsc.subcore_barrier()

        def scale(j, buf, s):
            @pl.loop(0, _C // 16)
            def grp(g):
                w16 = ewr[s, pl.ds(g * 16, 16)]
                for l in range(16):
                    w = w16[jnp.full((16,), l, _i32)]
                    e = g * 16 + l
                    for q in range(_D // 16):
                        sl = pl.ds(q * 16, 16)
                        buf[e, sl] = buf[e, sl] * w

        def start_src(j, s):
            pltpu.async_copy(src3_hbm.at[base + j], srcr.at[s], isems[s])

        def wait_src(j, s):
            pltpu.make_async_copy(src3_hbm.at[base + j], srcr.at[s],
                                  isems[s]).wait()

        def start_dst(j, s):
            pltpu.async_copy(dst3_hbm.at[base + j], dstr.at[s], dsems[s])

        def wait_dst(j, s):
            pltpu.make_async_copy(dst3_hbm.at[base + j], dstr.at[s],
                                  dsems[s]).wait()

        def start_ew(j, s):
            pltpu.async_copy(ew3_hbm.at[base + j], ewr.at[s], esems[s])

        def wait_ew(j, s):
            pltpu.make_async_copy(ew3_hbm.at[base + j], ewr.at[s],
                                  esems[s]).wait()

        def start_gather(j, s, b):
            pltpu.async_copy(h_hbm.at[srcr.at[s]], bufs[b], gsems[b])

        def wait_gather(j, s, b):
            pltpu.make_async_copy(h_hbm.at[srcr.at[s]], bufs[b],
                                  gsems[b]).wait()

        def start_scatter(j, s, b):
            pltpu.async_copy(bufs[b], acc_sh.at[dstr.at[s]], ssems[b],
                             add=True)

        def wait_scatter(j, s, b):
            pltpu.make_async_copy(bufs[b], acc_sh.at[dstr.at[s]],
                                  ssems[b]).wait()

        for j in (0, 1, 2):
            start_src(j, j)
        for j in (0, 1):
            start_dst(j, j)
            start_ew(j, j)
        wait_src(0, 0)
        start_gather(0, 0, 0)
        wait_src(1, 1)
        start_gather(1, 1, 1)

        def turn(jj, b, first=False, g2=True, s3=True, e2=True):
            b2 = (b + 2) % 3
            wait_gather(jj, b, b)
            wait_ew(jj, b)
            scale(jj, bufs[b], b)
            if e2:
                start_ew(jj + 2, b2)
            wait_dst(jj, b)
            start_scatter(jj, b, b)
            if not first:
                wait_scatter(jj - 1, b2, b2)
            if g2:
                wait_src(jj + 2, b2)
                start_gather(jj + 2, b2, b2)
                start_dst(jj + 2, b2)
            if s3:
                start_src(jj + 3, b)

        turn(0, 0, first=True)
        turn(1, 1)
        turn(2, 2)

        @pl.loop(3, kc - 4, step=3)
        def steady(j):
            for off in range(3):
                turn(j + off, off)

        turn(kc - 4, 0)
        turn(kc - 3, 1, s3=False)
        turn(kc - 2, 2, g2=False, s3=False, e2=False)
        turn(kc - 1, 0, g2=False, s3=False, e2=False)
        wait_scatter(kc - 1, 0, 0)

        plsc.subcore_barrier()
        for t in range(ncopies):
            rows = _C if t < nfull else rem
            pltpu.sync_copy(acc_sh.at[pl.ds(r0 + t * _C, rows), :],
                            buf0.at[pl.ds(0, rows), :])
            pltpu.sync_copy(buf0.at[pl.ds(0, rows), :],
                            sp_hbm.at[cid, pl.ds(r0 + t * _C, rows), :])

    f = pl.kernel(
        body,
        out_type=jax.ShapeDtypeStruct((_NC, _N, _D), _f32),
        mesh=_sc_mesh(),
        compiler_params=pltpu.CompilerParams(use_tc_tiling_on_sc=False),
        scratch_types=[
            pltpu.VMEM((3, _C), _i32),
            pltpu.VMEM((3, _C), _i32),
            pltpu.VMEM((3, _C), _f32),
            pltpu.VMEM((_C, _D), _f32),
            pltpu.VMEM((_C, _D), _f32),
            pltpu.VMEM((_C, _D), _f32),
            pltpu.VMEM_SHARED((_N, _D), _f32),
        ] + [pltpu.SemaphoreType.DMA] * 15,
    )
    return f(h, src3, dst3, ew3)


def _m1_body(x_ref, w_ref, d0_ref, d1_ref, hp_ref, dinv_ref):
    dinv = lax.rsqrt(1.0 + d0_ref[...] + d1_ref[...])
    h = jnp.dot(x_ref[...], w_ref[...], preferred_element_type=_f32)
    hp_ref[...] = h * dinv
    dinv_ref[...] = dinv


def _tc_m1(xp, W1, d0, d1):
    return pl.pallas_call(
        _m1_body,
        grid=(_G,),
        in_specs=[
            pl.BlockSpec((_BLK, _D), lambda i: (i, 0)),
            pl.BlockSpec((_D, _D), lambda i: (0, 0)),
            pl.BlockSpec((_BLK, 1), lambda i: (i, 0)),
            pl.BlockSpec((_BLK, 1), lambda i: (i, 0)),
        ],
        out_specs=[
            pl.BlockSpec((_BLK, _D), lambda i: (i, 0)),
            pl.BlockSpec((_BLK, 1), lambda i: (i, 0)),
        ],
        out_shape=[
            jax.ShapeDtypeStruct((_N, _D), _f32),
            jax.ShapeDtypeStruct((_N, 1), _f32),
        ],
    )(xp, W1, d0, d1)


def _m2_body(s_ref, hp_ref, dinv_ref, b_ref, w_ref, h2_ref):
    a = ((s_ref[0] + s_ref[1] + hp_ref[...]) * dinv_ref[...] + b_ref[...])
    a = jnp.maximum(a, 0.0)
    h2 = jnp.dot(a, w_ref[...], preferred_element_type=_f32)
    h2_ref[...] = h2 * dinv_ref[...]


def _tc_m2(s, hp, dinv, b, W2):
    return pl.pallas_call(
        _m2_body,
        grid=(_G,),
        in_specs=[
            pl.BlockSpec((_NC, _BLK, _D), lambda i: (0, i, 0)),
            pl.BlockSpec((_BLK, _D), lambda i: (i, 0)),
            pl.BlockSpec((_BLK, 1), lambda i: (i, 0)),
            pl.BlockSpec((1, _D), lambda i: (0, 0)),
            pl.BlockSpec((_D, _D), lambda i: (0, 0)),
        ],
        out_specs=pl.BlockSpec((_BLK, _D), lambda i: (i, 0)),
        out_shape=jax.ShapeDtypeStruct((_N, _D), _f32),
    )(s, hp, dinv, b, W2)


def _m3_body(s_ref, hp_ref, dinv_ref, b_ref, out_ref):
    out_ref[...] = ((s_ref[0] + s_ref[1] + hp_ref[...])
                    * dinv_ref[...] + b_ref[...])


def _tc_m3(s, hp, dinv, b):
    return pl.pallas_call(
        _m3_body,
        grid=(_G,),
        in_specs=[
            pl.BlockSpec((_NC, _BLK, _D), lambda i: (0, i, 0)),
            pl.BlockSpec((_BLK, _D), lambda i: (i, 0)),
            pl.BlockSpec((_BLK, 1), lambda i: (i, 0)),
            pl.BlockSpec((1, _D), lambda i: (0, 0)),
        ],
        out_specs=pl.BlockSpec((_BLK, _D), lambda i: (i, 0)),
        out_shape=jax.ShapeDtypeStruct((_N, _D), _f32),
    )(s, hp, dinv, b)


def kernel(x, edge_index, edge_weight, W1, b1, W2, b2):
    src = edge_index[0].astype(_i32)
    dst = edge_index[1].astype(_i32)
    ew = edge_weight.astype(_f32)
    pad = _EPAD - _E
    src3 = jnp.concatenate([src, jnp.zeros((pad,), _i32)]).reshape(_NCH, _C)
    dst3 = jnp.concatenate([dst, jnp.zeros((pad,), _i32)]).reshape(_NCH, _C)
    ew3 = jnp.concatenate([ew, jnp.zeros((pad,), _f32)]).reshape(_NCH, _C)

    degp = _deg_kernel(dst3, ew3).reshape(_NC, _NPAD)
    d0 = degp[0, :_N].reshape(_N, 1)
    d1 = degp[1, :_N].reshape(_N, 1)
    h1p, dinv = _tc_m1(x, W1, d0, d1)
    s1 = _edge_kernel(h1p, src3, dst3, ew3)
    h2p = _tc_m2(s1, h1p, dinv, b1.reshape(1, _D), W2)
    s2 = _edge_kernel(h2p, src3, dst3, ew3)
    outp = _tc_m3(s2, h2p, dinv, b2.reshape(1, _D))
    return outp


# speedup vs baseline: 1.0840x; 1.0163x over previous
import functools

import jax
import jax.numpy as jnp
from jax import lax
from jax.experimental import pallas as pl
from jax.experimental.pallas import tpu as pltpu
from jax.experimental.pallas import tpu_sc as plsc

_N = 10000
_E = 320000
_D = 128
_NC = 2
_NS = 16
_NW = _NC * _NS
_C = 128
_KA = 139
_KB = 19
_KM = max(_KA, _KB)
_NCH = _NS * (_KA + _KB)
_EPAD = _NCH * _C
_RPT = 632
_NPAD = _NS * _RPT
_RPE = _N // _NS
_BLK = 1000
_G = _N // _BLK

_f32 = jnp.float32
_i32 = jnp.int32


def _sc_mesh():
    return plsc.VectorSubcoreMesh(
        core_axis_name="c", subcore_axis_name="s",
        num_cores=_NC, num_subcores=_NS)


def _deg_kernel(dst3, ew3):
    def body(dst3_hbm, ew3_hbm, degp_hbm, dst_v, ew_v, zbuf, deg_sh):
        cid = lax.axis_index("c")
        sid = lax.axis_index("s")
        base = jnp.where(cid == 0, sid * _KA, _NS * _KA + sid * _KB)
        kc = jnp.where(cid == 0, _KA, _KB)
        r0 = pl.multiple_of(sid * _RPT, 8)

        @pl.loop(0, 40)
        def zz(t):
            zbuf[pl.ds(t * 16, 16)] = jnp.zeros((16,), _f32)

        pltpu.sync_copy(zbuf.at[pl.ds(0, _RPT)], deg_sh.at[pl.ds(r0, _RPT)])
        ld = jnp.minimum(base, _NCH - _KM)
        off = base - ld
        pltpu.sync_copy(dst3_hbm.at[pl.ds(ld, _KM), :], dst_v)
        pltpu.sync_copy(ew3_hbm.at[pl.ds(ld, _KM), :], ew_v)
        plsc.subcore_barrier()

        @pl.loop(0, kc)
        def chunk(j):
            pltpu.sync_copy(ew_v.at[off + j], deg_sh.at[dst_v.at[off + j]],
                            add=True)

        plsc.subcore_barrier()
        w0 = pl.multiple_of(cid * _NPAD + sid * _RPT, 8)
        pltpu.sync_copy(deg_sh.at[pl.ds(r0, _RPT)], zbuf.at[pl.ds(0, _RPT)])
        pltpu.sync_copy(zbuf.at[pl.ds(0, _RPT)], degp_hbm.at[pl.ds(w0, _RPT)])

    f = pl.kernel(
        body,
        out_type=jax.ShapeDtypeStruct((_NC * _NPAD,), _f32),
        mesh=_sc_mesh(),
        compiler_params=pltpu.CompilerParams(use_tc_tiling_on_sc=False),
        scratch_types=[
            pltpu.VMEM((_KM, _C), _i32),
            pltpu.VMEM((_KM, _C), _f32),
            pltpu.VMEM((640,), _f32),
            pltpu.VMEM_SHARED((_NPAD,), _f32),
        ],
    )
    return f(dst3, ew3)


def _edge_kernel(h, src3, dst3, ew3):
    def body(h_hbm, src3_hbm, dst3_hbm, ew3_hbm, sp_hbm,
             srcr, dstr, ewr, buf0, buf1, buf2, acc_sh,
             gsem0, gsem1, gsem2, ssem0, ssem1, ssem2,
             isem0, isem1, isem2, dsem0, dsem1, dsem2,
             esem0, esem1, esem2):
        cid = lax.axis_index("c")
        sid = lax.axis_index("s")
        base = jnp.where(cid == 0, sid * _KA, _NS * _KA + sid * _KB)
        kc = jnp.where(cid == 0, _KA, _KB)
        r0 = sid * _RPE
        bufs = (buf0, buf1, buf2)
        gsems = (gsem0, gsem1, gsem2)
        ssems = (ssem0, ssem1, ssem2)
        isems = (isem0, isem1, isem2)
        dsems = (dsem0, dsem1, dsem2)
        esems = (esem0, esem1, esem2)

        @pl.loop(0, _C)
        def zrow(e):
            for q in range(_D // 16):
                buf0[e, pl.ds(q * 16, 16)] = jnp.zeros((16,), _f32)

        nfull, rem = divmod(_RPE, _C)
        ncopies = nfull + (1 if rem else 0)
        for t in range(ncopies):
            rows = _C if t < nfull else rem
            pltpu.sync_copy(buf0.at[pl.ds(0, rows), :],
                            acc_sh.at[pl.ds(r0 + t * _C, rows), :])
        plsc.subcore_barrier()

        def scale(j, buf, s):
            @pl.loop(0, _C // 16)
            def grp(g):
                w16 = ewr[s, pl.ds(g * 16, 16)]
                for l in range(16):
                    w = w16[jnp.full((16,), l, _i32)]
                    e = g * 16 + l
                    for q in range(_D // 16):
                        sl = pl.ds(q * 16, 16)
                        buf[e, sl] = buf[e, sl] * w

        def start_src(j, s):
            pltpu.async_copy(src3_hbm.at[base + j], srcr.at[s], isems[s])

        def wait_src(j, s):
            pltpu.make_async_copy(src3_hbm.at[base + j], srcr.at[s],
                                  isems[s]).wait()

        def start_dst(j, s):
            pltpu.async_copy(dst3_hbm.at[base + j], dstr.at[s], dsems[s])

        def wait_dst(j, s):
            pltpu.make_async_copy(dst3_hbm.at[base + j], dstr.at[s],
                                  dsems[s]).wait()

        def start_ew(j, s):
            pltpu.async_copy(ew3_hbm.at[base + j], ewr.at[s], esems[s])

        def wait_ew(j, s):
            pltpu.make_async_copy(ew3_hbm.at[base + j], ewr.at[s],
                                  esems[s]).wait()

        def start_gather(j, s, b):
            pltpu.async_copy(h_hbm.at[srcr.at[s]], bufs[b], gsems[b])

        def wait_gather(j, s, b):
            pltpu.make_async_copy(h_hbm.at[srcr.at[s]], bufs[b],
                                  gsems[b]).wait()

        def start_scatter(j, s, b):
            pltpu.async_copy(bufs[b], acc_sh.at[dstr.at[s]], ssems[b],
                             add=True)

        def wait_scatter(j, s, b):
            pltpu.make_async_copy(bufs[b], acc_sh.at[dstr.at[s]],
                                  ssems[b]).wait()

        for j in (0, 1, 2):
            start_src(j, j)
        for j in (0, 1):
            start_dst(j, j)
            start_ew(j, j)
        wait_src(0, 0)
        start_gather(0, 0, 0)
        wait_src(1, 1)
        start_gather(1, 1, 1)

        def turn(jj, b, first=False, g2=True, s3=True, e2=True):
            b2 = (b + 2) % 3
            wait_gather(jj, b, b)
            wait_ew(jj, b)
            scale(jj, bufs[b], b)
            if e2:
                start_ew(jj + 2, b2)
            wait_dst(jj, b)
            start_scatter(jj, b, b)
            if not first:
                wait_scatter(jj - 1, b2, b2)
            if g2:
                wait_src(jj + 2, b2)
                start_gather(jj + 2, b2, b2)
                start_dst(jj + 2, b2)
            if s3:
                start_src(jj + 3, b)

        turn(0, 0, first=True)
        turn(1, 1)
        turn(2, 2)

        @pl.loop(3, kc - 4, step=3)
        def steady(j):
            for off in range(3):
                turn(j + off, off)

        turn(kc - 4, 0)
        turn(kc - 3, 1, s3=False)
        turn(kc - 2, 2, g2=False, s3=False, e2=False)
        turn(kc - 1, 0, g2=False, s3=False, e2=False)
        wait_scatter(kc - 1, 0, 0)

        plsc.subcore_barrier()
        for t in range(ncopies):
            rows = _C if t < nfull else rem
            pltpu.sync_copy(acc_sh.at[pl.ds(r0 + t * _C, rows), :],
                            buf0.at[pl.ds(0, rows), :])
            pltpu.sync_copy(buf0.at[pl.ds(0, rows), :],
                            sp_hbm.at[cid, pl.ds(r0 + t * _C, rows), :])

    f = pl.kernel(
        body,
        out_type=jax.ShapeDtypeStruct((_NC, _N, _D), _f32),
        mesh=_sc_mesh(),
        compiler_params=pltpu.CompilerParams(use_tc_tiling_on_sc=False),
        scratch_types=[
            pltpu.VMEM((3, _C), _i32),
            pltpu.VMEM((3, _C), _i32),
            pltpu.VMEM((3, _C), _f32),
            pltpu.VMEM((_C, _D), _f32),
            pltpu.VMEM((_C, _D), _f32),
            pltpu.VMEM((_C, _D), _f32),
            pltpu.VMEM_SHARED((_N, _D), _f32),
        ] + [pltpu.SemaphoreType.DMA] * 15,
    )
    return f(h, src3, dst3, ew3)


def _m1_body(x_ref, w_ref, d0_ref, d1_ref, hp_ref, dinv_ref):
    dinv = lax.rsqrt(1.0 + d0_ref[...] + d1_ref[...])
    h = jnp.dot(x_ref[...], w_ref[...], preferred_element_type=_f32)
    hp_ref[...] = h * dinv
    dinv_ref[...] = dinv


def _tc_m1(xp, W1, d0, d1):
    return pl.pallas_call(
        _m1_body,
        grid=(_G,),
        in_specs=[
            pl.BlockSpec((_BLK, _D), lambda i: (i, 0)),
            pl.BlockSpec((_D, _D), lambda i: (0, 0)),
            pl.BlockSpec((_BLK, 1), lambda i: (i, 0)),
            pl.BlockSpec((_BLK, 1), lambda i: (i, 0)),
        ],
        out_specs=[
            pl.BlockSpec((_BLK, _D), lambda i: (i, 0)),
            pl.BlockSpec((_BLK, 1), lambda i: (i, 0)),
        ],
        out_shape=[
            jax.ShapeDtypeStruct((_N, _D), _f32),
            jax.ShapeDtypeStruct((_N, 1), _f32),
        ],
    )(xp, W1, d0, d1)


def _m2_body(s_ref, hp_ref, dinv_ref, b_ref, w_ref, h2_ref):
    a = ((s_ref[0] + s_ref[1] + hp_ref[...]) * dinv_ref[...] + b_ref[...])
    a = jnp.maximum(a, 0.0)
    h2 = jnp.dot(a, w_ref[...], preferred_element_type=_f32)
    h2_ref[...] = h2 * dinv_ref[...]


def _tc_m2(s, hp, dinv, b, W2):
    return pl.pallas_call(
        _m2_body,
        grid=(_G,),
        in_specs=[
            pl.BlockSpec((_NC, _BLK, _D), lambda i: (0, i, 0)),
            pl.BlockSpec((_BLK, _D), lambda i: (i, 0)),
            pl.BlockSpec((_BLK, 1), lambda i: (i, 0)),
            pl.BlockSpec((1, _D), lambda i: (0, 0)),
            pl.BlockSpec((_D, _D), lambda i: (0, 0)),
        ],
        out_specs=pl.BlockSpec((_BLK, _D), lambda i: (i, 0)),
        out_shape=jax.ShapeDtypeStruct((_N, _D), _f32),
    )(s, hp, dinv, b, W2)


def _m3_body(s_ref, hp_ref, dinv_ref, b_ref, out_ref):
    out_ref[...] = ((s_ref[0] + s_ref[1] + hp_ref[...])
                    * dinv_ref[...] + b_ref[...])


def _tc_m3(s, hp, dinv, b):
    return pl.pallas_call(
        _m3_body,
        grid=(_G,),
        in_specs=[
            pl.BlockSpec((_NC, _BLK, _D), lambda i: (0, i, 0)),
            pl.BlockSpec((_BLK, _D), lambda i: (i, 0)),
            pl.BlockSpec((_BLK, 1), lambda i: (i, 0)),
            pl.BlockSpec((1, _D), lambda i: (0, 0)),
        ],
        out_specs=pl.BlockSpec((_BLK, _D), lambda i: (i, 0)),
        out_shape=jax.ShapeDtypeStruct((_N, _D), _f32),
    )(s, hp, dinv, b)


def kernel(x, edge_index, edge_weight, W1, b1, W2, b2):
    src = edge_index[0].astype(_i32)
    dst = edge_index[1].astype(_i32)
    ew = edge_weight.astype(_f32)
    pad = _EPAD - _E
    src3 = jnp.concatenate([src, jnp.zeros((pad,), _i32)]).reshape(_NCH, _C)
    dst3 = jnp.concatenate([dst, jnp.zeros((pad,), _i32)]).reshape(_NCH, _C)
    ew3 = jnp.concatenate([ew, jnp.zeros((pad,), _f32)]).reshape(_NCH, _C)

    degp = _deg_kernel(dst3, ew3).reshape(_NC, _NPAD)
    d0 = degp[0, :_N].reshape(_N, 1)
    d1 = degp[1, :_N].reshape(_N, 1)
    h1p, dinv = _tc_m1(x, W1, d0, d1)
    s1 = _edge_kernel(h1p, src3, dst3, ew3)
    h2p = _tc_m2(s1, h1p, dinv, b1.reshape(1, _D), W2)
    s2 = _edge_kernel(h2p, src3, dst3, ew3)
    outp = _tc_m3(s2, h2p, dinv, b2.reshape(1, _D))
    return outp
